# Initial kernel scaffold; baseline (speedup 1.0000x reference)
import jax
import jax.numpy as jnp
from jax.experimental import pallas as pl


def kernel(node_pos, edge_index, edge_weight):
    raise NotImplementedError("write your pallas kernel here")


# speedup vs baseline: 1.8024x; 1.8024x over previous
import functools

import jax
import jax.numpy as jnp
from jax import lax
from jax.experimental import pallas as pl
from jax.experimental.pallas import tpu as pltpu
from jax.experimental.pallas import tpu_sc as plsc

_A = 1.576943460405378
_B = 0.8950608781227859
_P = 65536
_NEG_PER_EDGE = 5
_GAMMA = 1.0
_DIM = 64

_NC, _NS, _L = 2, 16, 16
_NW = _NC * _NS
_NNEG = _P * _NEG_PER_EDGE
_POS_PER_W = _P // _NW
_NEG_PER_W = _NNEG // _NW
_CHUNK = 128
_POS_CHUNKS = _POS_PER_W // _CHUNK
_NEG_CHUNKS = _NEG_PER_W // _CHUNK
_BLKS = _CHUNK // _L

_LN2 = 0.6931471805599453
_MAX_TERM = 9.210340371976184
_SQRT2 = 1.4142135623730951


def _vlog(x):
    bits = plsc.bitcast(x, jnp.int32)
    e = (bits >> 23) - 127
    m = plsc.bitcast((bits & 0x007FFFFF) | 0x3F800000, jnp.float32)
    big = m > _SQRT2
    m = jnp.where(big, m * 0.5, m)
    e = jnp.where(big, e + 1, e)
    z = (m - 1.0) / (m + 1.0)
    z2 = z * z
    poly = 1.0 + z2 * (1.0 / 3.0 + z2 * (1.0 / 5.0 + z2 * (1.0 / 7.0)))
    return e.astype(jnp.float32) * _LN2 + 2.0 * z * poly


def _loss_terms(s, is_pos):
    t = s + 1e-12
    u = _A * jnp.exp(_B * _vlog(t))
    if is_pos:
        v = 1.0 + u
    else:
        v = 1.0 + 1.0 / u
    return jnp.minimum(_vlog(v), _MAX_TERM)


def _chunk_loss(i_rows, j_rows, acc, is_pos):
    iota = lax.iota(jnp.int32, _L)

    def blk(b, acc):
        row = b * _L + iota
        s = jnp.zeros((_L,), jnp.float32)
        for d in range(_DIM):
            col = jnp.full((_L,), d, jnp.int32)
            df = plsc.load_gather(i_rows, [row, col]) - plsc.load_gather(
                j_rows, [row, col])
            s = s + df * df
        return acc + _loss_terms(s, is_pos)

    return lax.fori_loop(0, _BLKS, blk, acc)


def _sc_body(node_hbm, e0_hbm, e1_hbm, pos_idx_hbm, neg_src_hbm, neg_dst_hbm,
             out_hbm, idx_buf, src_buf, dst_buf, i_rows, j_rows, stage, sem):
    n_nodes = node_hbm.shape[0]
    wid = lax.axis_index("s") * _NC + lax.axis_index("c")

    def pos_chunk(c, acc):
        base = wid * _POS_PER_W + c * _CHUNK
        pltpu.sync_copy(pos_idx_hbm.at[pl.ds(base, _CHUNK)], idx_buf)
        a = pltpu.async_copy(e0_hbm.at[idx_buf], src_buf, sem)
        b = pltpu.async_copy(e1_hbm.at[idx_buf], dst_buf, sem)
        a.wait()
        b.wait()
        a = pltpu.async_copy(node_hbm.at[src_buf], i_rows, sem)
        b = pltpu.async_copy(node_hbm.at[dst_buf], j_rows, sem)
        a.wait()
        b.wait()
        return _chunk_loss(i_rows, j_rows, acc, True)

    def neg_chunk(c, acc):
        base = wid * _NEG_PER_W + c * _CHUNK
        a = pltpu.async_copy(neg_src_hbm.at[pl.ds(base, _CHUNK)], src_buf, sem)
        b = pltpu.async_copy(neg_dst_hbm.at[pl.ds(base, _CHUNK)], dst_buf, sem)
        a.wait()
        b.wait()

        def fix(k, carry):
            vs = src_buf[pl.ds(k * _L, _L)]
            vd = dst_buf[pl.ds(k * _L, _L)]
            vd1 = vd + 1
            vd1 = jnp.where(vd1 == n_nodes, 0, vd1)
            dst_buf[pl.ds(k * _L, _L)] = jnp.where(vd == vs, vd1, vd)
            return carry

        lax.fori_loop(0, _BLKS, fix, 0)
        a = pltpu.async_copy(node_hbm.at[src_buf], i_rows, sem)
        b = pltpu.async_copy(node_hbm.at[dst_buf], j_rows, sem)
        a.wait()
        b.wait()
        return _chunk_loss(i_rows, j_rows, acc, False)

    zero = jnp.zeros((_L,), jnp.float32)
    acc_pos = lax.fori_loop(0, _POS_CHUNKS, pos_chunk, zero)
    acc_neg = lax.fori_loop(0, _NEG_CHUNKS, neg_chunk, zero)
    stage[0] = acc_pos
    stage[1] = acc_neg
    pltpu.sync_copy(stage, out_hbm.at[wid])


@jax.jit
def _sc_loss(node_pos, e0, e1, pos_idx, neg_src, neg_dst):
    mesh = plsc.VectorSubcoreMesh(core_axis_name="c", subcore_axis_name="s")
    f = functools.partial(
        pl.kernel,
        out_type=jax.ShapeDtypeStruct((_NW, 2, _L), jnp.float32),
        mesh=mesh,
        compiler_params=pltpu.CompilerParams(
            needs_layout_passes=False, use_tc_tiling_on_sc=False),
        scratch_types=[
            pltpu.VMEM((_CHUNK,), jnp.int32),
            pltpu.VMEM((_CHUNK,), jnp.int32),
            pltpu.VMEM((_CHUNK,), jnp.int32),
            pltpu.VMEM((_CHUNK, _DIM), jnp.float32),
            pltpu.VMEM((_CHUNK, _DIM), jnp.float32),
            pltpu.VMEM((2, _L), jnp.float32),
            pltpu.SemaphoreType.DMA,
        ],
    )(_sc_body)
    return f(node_pos, e0, e1, pos_idx, neg_src, neg_dst)


def kernel(node_pos, edge_index, edge_weight):
    n_nodes = node_pos.shape[0]
    n_edges = edge_index.shape[1]
    w = jnp.clip(edge_weight, 1e-12, None)
    p = w / w.sum()
    key = jax.random.key(42)
    kpos, kneg = jax.random.split(key)
    pos_idx = jax.random.choice(kpos, n_edges, shape=(_P,), replace=True,
                                p=p).astype(jnp.int32)
    kn1, kn2 = jax.random.split(kneg)
    neg_src = jax.random.randint(kn1, (_NNEG,), 0, n_nodes, dtype=jnp.int32)
    neg_dst = jax.random.randint(kn2, (_NNEG,), 0, n_nodes, dtype=jnp.int32)

    parts = _sc_loss(node_pos, edge_index[0], edge_index[1], pos_idx,
                     neg_src, neg_dst)
    attraction = jnp.sum(parts[:, 0, :]) / _P
    repulsion = jnp.sum(parts[:, 1, :]) / _NNEG
    return attraction + _GAMMA * repulsion


# speedup vs baseline: 1.9659x; 1.0907x over previous
import functools

import jax
import jax.numpy as jnp
from jax import lax
from jax.experimental import pallas as pl
from jax.experimental.pallas import tpu as pltpu
from jax.experimental.pallas import tpu_sc as plsc

_A = 1.576943460405378
_B = 0.8950608781227859
_P = 65536
_NEG_PER_EDGE = 5
_GAMMA = 1.0
_DIM = 64

_NC, _NS, _L = 2, 16, 16
_NW = _NC * _NS
_NNEG = _P * _NEG_PER_EDGE
_POS_PER_W = _P // _NW
_NEG_PER_W = _NNEG // _NW
_PAIRS_PER_W = _POS_PER_W + _NEG_PER_W
_CHUNK = 128
_POS_CHUNKS = _POS_PER_W // _CHUNK
_NCHUNKS = _PAIRS_PER_W // _CHUNK
_NB = 3
_BLKS = _CHUNK // _L

_LN2 = 0.6931471805599453
_MAX_TERM = 9.210340371976184
_SQRT2 = 1.4142135623730951


def _vlog(x):
    bits = plsc.bitcast(x, jnp.int32)
    e = (bits >> 23) - 127
    m = plsc.bitcast((bits & 0x007FFFFF) | 0x3F800000, jnp.float32)
    big = m > _SQRT2
    m = jnp.where(big, m * 0.5, m)
    e = jnp.where(big, e + 1, e)
    z = (m - 1.0) / (m + 1.0)
    z2 = z * z
    poly = 1.0 + z2 * (1.0 / 3.0 + z2 * (1.0 / 5.0 + z2 * (1.0 / 7.0)))
    return e.astype(jnp.float32) * _LN2 + 2.0 * z * poly


def _sc_body(node_hbm, e0_hbm, e1_hbm, pos_idx_hbm, neg_src_hbm, neg_dst_hbm,
             out_hbm, posid, src_all, dst_all, rows_i, rows_j, stage,
             sem_p, sems_i, sems_j):
    n_nodes = node_hbm.shape[0]
    wid = lax.axis_index("s") * _NC + lax.axis_index("c")

    pltpu.sync_copy(pos_idx_hbm.at[pl.ds(wid * _POS_PER_W, _POS_PER_W)], posid)
    descs = []
    for k in range(_POS_CHUNKS):
        sl = pl.ds(k * _CHUNK, _CHUNK)
        descs.append(pltpu.async_copy(e0_hbm.at[posid.at[sl]],
                                      src_all.at[sl], sem_p))
        descs.append(pltpu.async_copy(e1_hbm.at[posid.at[sl]],
                                      dst_all.at[sl], sem_p))
    nsl = pl.ds(_POS_PER_W, _NEG_PER_W)
    pltpu.sync_copy(neg_src_hbm.at[pl.ds(wid * _NEG_PER_W, _NEG_PER_W)],
                    src_all.at[nsl])
    pltpu.sync_copy(neg_dst_hbm.at[pl.ds(wid * _NEG_PER_W, _NEG_PER_W)],
                    dst_all.at[nsl])
    for d in descs:
        d.wait()

    def fix(k, carry):
        sl = pl.ds(_POS_PER_W + k * _L, _L)
        vs = src_all[sl]
        vd = dst_all[sl]
        vd1 = vd + 1
        vd1 = jnp.where(vd1 == n_nodes, 0, vd1)
        dst_all[sl] = jnp.where(vd == vs, vd1, vd)
        return carry

    lax.fori_loop(0, _NEG_PER_W // _L, fix, 0)

    def issue(c, b):
        sl = pl.ds(c * _CHUNK, _CHUNK)
        rsl = pl.ds(b * _CHUNK, _CHUNK)
        pltpu.async_copy(node_hbm.at[src_all.at[sl]], rows_i.at[rsl],
                         sems_i.at[b])
        pltpu.async_copy(node_hbm.at[dst_all.at[sl]], rows_j.at[rsl],
                         sems_j.at[b])

    def wait(c, b):
        sl = pl.ds(c * _CHUNK, _CHUNK)
        rsl = pl.ds(b * _CHUNK, _CHUNK)
        pltpu.make_async_copy(node_hbm.at[src_all.at[sl]], rows_i.at[rsl],
                              sems_i.at[b]).wait()
        pltpu.make_async_copy(node_hbm.at[dst_all.at[sl]], rows_j.at[rsl],
                              sems_j.at[b]).wait()

    iota = lax.iota(jnp.int32, _L)

    def chunk_loss(b, is_pos_vec, accs):
        acc_pos, acc_neg = accs

        def blk(kb, accs2):
            acc_pos2, acc_neg2 = accs2
            row = b * _CHUNK + kb * _L + iota
            s = jnp.zeros((_L,), jnp.float32)
            for d in range(_DIM):
                col = jnp.full((_L,), d, jnp.int32)
                df = plsc.load_gather(rows_i, [row, col]) - plsc.load_gather(
                    rows_j, [row, col])
                s = s + df * df
            t = s + 1e-12
            u = _A * jnp.exp(_B * _vlog(t))
            v = jnp.where(is_pos_vec, 1.0 + u, 1.0 + 1.0 / u)
            term = jnp.minimum(_vlog(v), _MAX_TERM)
            zero = jnp.zeros((_L,), jnp.float32)
            acc_pos2 = acc_pos2 + jnp.where(is_pos_vec, term, zero)
            acc_neg2 = acc_neg2 + jnp.where(is_pos_vec, zero, term)
            return acc_pos2, acc_neg2

        return lax.fori_loop(0, _BLKS, blk, (acc_pos, acc_neg))

    for b in range(_NB):
        issue(b, b)

    zero = jnp.zeros((_L,), jnp.float32)

    def outer(g, accs):
        for b in range(_NB):
            c = g * _NB + b
            wait(c, b)
            is_pos_vec = jnp.full((_L,), c, jnp.int32) < _POS_CHUNKS
            accs = chunk_loss(b, is_pos_vec, accs)

            @pl.when(c + _NB < _NCHUNKS)
            def _():
                issue(c + _NB, b)

        return accs

    acc_pos, acc_neg = lax.fori_loop(0, _NCHUNKS // _NB, outer, (zero, zero))
    stage[0] = acc_pos
    stage[1] = acc_neg
    pltpu.sync_copy(stage, out_hbm.at[wid])


@jax.jit
def _sc_loss(node_pos, e0, e1, pos_idx, neg_src, neg_dst):
    mesh = plsc.VectorSubcoreMesh(core_axis_name="c", subcore_axis_name="s")
    f = functools.partial(
        pl.kernel,
        out_type=jax.ShapeDtypeStruct((_NW, 2, _L), jnp.float32),
        mesh=mesh,
        compiler_params=pltpu.CompilerParams(
            needs_layout_passes=False, use_tc_tiling_on_sc=False),
        scratch_types=[
            pltpu.VMEM((_POS_PER_W,), jnp.int32),
            pltpu.VMEM((_PAIRS_PER_W,), jnp.int32),
            pltpu.VMEM((_PAIRS_PER_W,), jnp.int32),
            pltpu.VMEM((_NB * _CHUNK, _DIM), jnp.float32),
            pltpu.VMEM((_NB * _CHUNK, _DIM), jnp.float32),
            pltpu.VMEM((2, _L), jnp.float32),
            pltpu.SemaphoreType.DMA,
            pltpu.SemaphoreType.DMA((_NB,)),
            pltpu.SemaphoreType.DMA((_NB,)),
        ],
    )(_sc_body)
    return f(node_pos, e0, e1, pos_idx, neg_src, neg_dst)


def kernel(node_pos, edge_index, edge_weight):
    n_nodes = node_pos.shape[0]
    n_edges = edge_index.shape[1]
    w = jnp.clip(edge_weight, 1e-12, None)
    p = w / w.sum()
    key = jax.random.key(42)
    kpos, kneg = jax.random.split(key)
    pos_idx = jax.random.choice(kpos, n_edges, shape=(_P,), replace=True,
                                p=p).astype(jnp.int32)
    kn1, kn2 = jax.random.split(kneg)
    neg_src = jax.random.randint(kn1, (_NNEG,), 0, n_nodes, dtype=jnp.int32)
    neg_dst = jax.random.randint(kn2, (_NNEG,), 0, n_nodes, dtype=jnp.int32)

    parts = _sc_loss(node_pos, edge_index[0], edge_index[1], pos_idx,
                     neg_src, neg_dst)
    attraction = jnp.sum(parts[:, 0, :]) / _P
    repulsion = jnp.sum(parts[:, 1, :]) / _NNEG
    return attraction + _GAMMA * repulsion


# speedup vs baseline: 2.9617x; 1.5065x over previous
import functools

import jax
import jax.numpy as jnp
from jax import lax
from jax.experimental import pallas as pl
from jax.experimental.pallas import tpu as pltpu
from jax.experimental.pallas import tpu_sc as plsc

_A = 1.576943460405378
_B = 0.8950608781227859
_P = 65536
_NEG_PER_EDGE = 5
_GAMMA = 1.0
_DIM = 64

_NC, _NS, _L = 2, 16, 16
_NW = _NC * _NS
_NNEG = _P * _NEG_PER_EDGE
_POS_PER_W = _P // _NW
_NEG_PER_W = _NNEG // _NW
_PAIRS_PER_W = _POS_PER_W + _NEG_PER_W
_CHUNK = 128
_POS_CHUNKS = _POS_PER_W // _CHUNK
_NCHUNKS = _PAIRS_PER_W // _CHUNK
_NB = 3
_BLKS = _CHUNK // _L

_LN2 = 0.6931471805599453
_MAX_TERM = 9.210340371976184
_SQRT2 = 1.4142135623730951


def _vlog(x):
    bits = plsc.bitcast(x, jnp.int32)
    e = (bits >> 23) - 127
    m = plsc.bitcast((bits & 0x007FFFFF) | 0x3F800000, jnp.float32)
    big = m > _SQRT2
    m = jnp.where(big, m * 0.5, m)
    e = jnp.where(big, e + 1, e)
    z = (m - 1.0) / (m + 1.0)
    z2 = z * z
    poly = 1.0 + z2 * (1.0 / 3.0 + z2 * (1.0 / 5.0 + z2 * (1.0 / 7.0)))
    return e.astype(jnp.float32) * _LN2 + 2.0 * z * poly


def _sc_body(node_hbm, e0_hbm, e1_hbm, pos_idx_hbm, neg_src_hbm, neg_dst_hbm,
             out_hbm, posid, src_all, dst_all, rows_i, rows_j, stage,
             sem_p, sems_i, sems_j):
    n_nodes = node_hbm.shape[0]
    wid = lax.axis_index("s") * _NC + lax.axis_index("c")

    pltpu.sync_copy(pos_idx_hbm.at[pl.ds(wid * _POS_PER_W, _POS_PER_W)], posid)
    descs = []
    for k in range(_POS_CHUNKS):
        sl = pl.ds(k * _CHUNK, _CHUNK)
        descs.append(pltpu.async_copy(e0_hbm.at[posid.at[sl]],
                                      src_all.at[sl], sem_p))
        descs.append(pltpu.async_copy(e1_hbm.at[posid.at[sl]],
                                      dst_all.at[sl], sem_p))
    nsl = pl.ds(_POS_PER_W, _NEG_PER_W)
    pltpu.sync_copy(neg_src_hbm.at[pl.ds(wid * _NEG_PER_W, _NEG_PER_W)],
                    src_all.at[nsl])
    pltpu.sync_copy(neg_dst_hbm.at[pl.ds(wid * _NEG_PER_W, _NEG_PER_W)],
                    dst_all.at[nsl])
    for d in descs:
        d.wait()

    def fix(k, carry):
        sl = pl.ds(_POS_PER_W + k * _L, _L)
        vs = src_all[sl]
        vd = dst_all[sl]
        vd1 = vd + 1
        vd1 = jnp.where(vd1 == n_nodes, 0, vd1)
        dst_all[sl] = jnp.where(vd == vs, vd1, vd)
        return carry

    lax.fori_loop(0, _NEG_PER_W // _L, fix, 0)

    def issue(c, b):
        sl = pl.ds(c * _CHUNK, _CHUNK)
        rsl = pl.ds(b * _CHUNK, _CHUNK)
        pltpu.async_copy(node_hbm.at[src_all.at[sl]], rows_i.at[rsl],
                         sems_i.at[b])
        pltpu.async_copy(node_hbm.at[dst_all.at[sl]], rows_j.at[rsl],
                         sems_j.at[b])

    def wait(c, b):
        sl = pl.ds(c * _CHUNK, _CHUNK)
        rsl = pl.ds(b * _CHUNK, _CHUNK)
        pltpu.make_async_copy(node_hbm.at[src_all.at[sl]], rows_i.at[rsl],
                              sems_i.at[b]).wait()
        pltpu.make_async_copy(node_hbm.at[dst_all.at[sl]], rows_j.at[rsl],
                              sems_j.at[b]).wait()

    iota = lax.iota(jnp.int32, _L)

    def chunk_loss(b, is_pos_vec, accs):
        acc_pos, acc_neg = accs

        def blk(kb, accs2):
            acc_pos2, acc_neg2 = accs2
            row = b * _CHUNK + kb * _L + iota
            s = jnp.zeros((_L,), jnp.float32)
            for d in range(_DIM):
                col = (iota + d) & (_DIM - 1)
                df = plsc.load_gather(rows_i, [row, col]) - plsc.load_gather(
                    rows_j, [row, col])
                s = s + df * df
            t = s + 1e-12
            u = _A * jnp.exp(_B * _vlog(t))
            v = jnp.where(is_pos_vec, 1.0 + u, 1.0 + 1.0 / u)
            term = jnp.minimum(_vlog(v), _MAX_TERM)
            zero = jnp.zeros((_L,), jnp.float32)
            acc_pos2 = acc_pos2 + jnp.where(is_pos_vec, term, zero)
            acc_neg2 = acc_neg2 + jnp.where(is_pos_vec, zero, term)
            return acc_pos2, acc_neg2

        return lax.fori_loop(0, _BLKS, blk, (acc_pos, acc_neg))

    for b in range(_NB):
        issue(b, b)

    zero = jnp.zeros((_L,), jnp.float32)

    def outer(g, accs):
        for b in range(_NB):
            c = g * _NB + b
            wait(c, b)
            is_pos_vec = jnp.full((_L,), c, jnp.int32) < _POS_CHUNKS
            accs = chunk_loss(b, is_pos_vec, accs)

            @pl.when(c + _NB < _NCHUNKS)
            def _():
                issue(c + _NB, b)

        return accs

    acc_pos, acc_neg = lax.fori_loop(0, _NCHUNKS // _NB, outer, (zero, zero))
    stage[0] = acc_pos
    stage[1] = acc_neg
    pltpu.sync_copy(stage, out_hbm.at[wid])


@jax.jit
def _sc_loss(node_pos, e0, e1, pos_idx, neg_src, neg_dst):
    mesh = plsc.VectorSubcoreMesh(core_axis_name="c", subcore_axis_name="s")
    f = functools.partial(
        pl.kernel,
        out_type=jax.ShapeDtypeStruct((_NW, 2, _L), jnp.float32),
        mesh=mesh,
        compiler_params=pltpu.CompilerParams(
            needs_layout_passes=False, use_tc_tiling_on_sc=False),
        scratch_types=[
            pltpu.VMEM((_POS_PER_W,), jnp.int32),
            pltpu.VMEM((_PAIRS_PER_W,), jnp.int32),
            pltpu.VMEM((_PAIRS_PER_W,), jnp.int32),
            pltpu.VMEM((_NB * _CHUNK, _DIM), jnp.float32),
            pltpu.VMEM((_NB * _CHUNK, _DIM), jnp.float32),
            pltpu.VMEM((2, _L), jnp.float32),
            pltpu.SemaphoreType.DMA,
            pltpu.SemaphoreType.DMA((_NB,)),
            pltpu.SemaphoreType.DMA((_NB,)),
        ],
    )(_sc_body)
    return f(node_pos, e0, e1, pos_idx, neg_src, neg_dst)


def kernel(node_pos, edge_index, edge_weight):
    n_nodes = node_pos.shape[0]
    n_edges = edge_index.shape[1]
    w = jnp.clip(edge_weight, 1e-12, None)
    p = w / w.sum()
    key = jax.random.key(42)
    kpos, kneg = jax.random.split(key)
    pos_idx = jax.random.choice(kpos, n_edges, shape=(_P,), replace=True,
                                p=p).astype(jnp.int32)
    kn1, kn2 = jax.random.split(kneg)
    neg_src = jax.random.randint(kn1, (_NNEG,), 0, n_nodes, dtype=jnp.int32)
    neg_dst = jax.random.randint(kn2, (_NNEG,), 0, n_nodes, dtype=jnp.int32)

    parts = _sc_loss(node_pos, edge_index[0], edge_index[1], pos_idx,
                     neg_src, neg_dst)
    attraction = jnp.sum(parts[:, 0, :]) / _P
    repulsion = jnp.sum(parts[:, 1, :]) / _NNEG
    return attraction + _GAMMA * repulsion


# speedup vs baseline: 7.8860x; 2.6627x over previous
import functools

import jax
import jax.numpy as jnp
from jax import lax
from jax.experimental import pallas as pl
from jax.experimental.pallas import tpu as pltpu
from jax.experimental.pallas import tpu_sc as plsc

_A = 1.576943460405378
_B = 0.8950608781227859
_P = 65536
_NEG_PER_EDGE = 5
_GAMMA = 1.0
_DIM = 64

_NC, _NS, _L = 2, 16, 16
_NW = _NC * _NS
_NNEG = _P * _NEG_PER_EDGE
_POS_PER_W = _P // _NW
_NEG_PER_W = _NNEG // _NW
_PAIRS_PER_W = _POS_PER_W + _NEG_PER_W
_CHUNK = 128
_POS_CHUNKS = _POS_PER_W // _CHUNK
_NCHUNKS = _PAIRS_PER_W // _CHUNK
_NB = 3
_BLKS = _CHUNK // _L

_COARSE_STEP = 64
_LOCAL_LEVELS = 14
_HBM_LEVELS = 6

_LN2 = 0.6931471805599453
_MAX_TERM = 9.210340371976184
_SQRT2 = 1.4142135623730951


def _vlog(x):
    bits = plsc.bitcast(x, jnp.int32)
    e = (bits >> 23) - 127
    m = plsc.bitcast((bits & 0x007FFFFF) | 0x3F800000, jnp.float32)
    big = m > _SQRT2
    m = jnp.where(big, m * 0.5, m)
    e = jnp.where(big, e + 1, e)
    z = (m - 1.0) / (m + 1.0)
    z2 = z * z
    poly = 1.0 + z2 * (1.0 / 3.0 + z2 * (1.0 / 5.0 + z2 * (1.0 / 7.0)))
    return e.astype(jnp.float32) * _LN2 + 2.0 * z * poly


def _sc_body(node_hbm, e0_hbm, e1_hbm, pcuml_hbm, coarse_hbm, r_hbm,
             neg_src_hbm, neg_dst_hbm, out_hbm,
             posid, src_all, dst_all, rows_i, rows_j, stage,
             coarse, rbuf, lbuf, rbnd, valbuf,
             sem_p, sem_n, sems_i, sems_j):
    n_nodes = node_hbm.shape[0]
    n_edges = pcuml_hbm.shape[0]
    n_coarse = (n_edges + _COARSE_STEP - 1) // _COARSE_STEP
    wid = lax.axis_index("s") * _NC + lax.axis_index("c")
    iota = lax.iota(jnp.int32, _L)
    n_blk = _POS_PER_W // _L

    nsl = pl.ds(_POS_PER_W, _NEG_PER_W)
    dna = pltpu.async_copy(
        neg_src_hbm.at[pl.ds(wid * _NEG_PER_W, _NEG_PER_W)],
        src_all.at[nsl], sem_n)
    dnb = pltpu.async_copy(
        neg_dst_hbm.at[pl.ds(wid * _NEG_PER_W, _NEG_PER_W)],
        dst_all.at[nsl], sem_n)

    pltpu.sync_copy(coarse_hbm.at[pl.ds(0, n_coarse)], coarse)
    pltpu.sync_copy(r_hbm.at[pl.ds(wid * _POS_PER_W, _POS_PER_W)], rbuf)

    def local_search(b, carry):
        sl = pl.ds(b * _L, _L)
        rv = rbuf[sl]
        lo = jnp.zeros((_L,), jnp.int32)
        hi = jnp.full((_L,), n_coarse - 1, jnp.int32)
        for _ in range(_LOCAL_LEVELS):
            mid = (lo + hi) >> 1
            ge = plsc.load_gather(coarse, [mid]) >= rv
            hi = jnp.where(ge, mid, hi)
            lo = jnp.where(ge, lo, mid + 1)
        jc = jnp.minimum(lo, n_coarse - 1)
        j = jnp.where(plsc.load_gather(coarse, [jc]) >= rv, jc, jc + 1)
        lo0 = jnp.maximum(j * _COARSE_STEP - (_COARSE_STEP - 1), 0)
        hi0 = jnp.minimum(j * _COARSE_STEP, n_edges - 1)
        lbuf[sl] = lo0
        rbnd[sl] = hi0
        return carry

    lax.fori_loop(0, n_blk, local_search, 0)

    for _ in range(_HBM_LEVELS):
        def mids(b, carry):
            sl = pl.ds(b * _L, _L)
            posid[sl] = (lbuf[sl] + rbnd[sl]) >> 1
            return carry

        lax.fori_loop(0, n_blk, mids, 0)
        descs = []
        for k in range(_POS_CHUNKS):
            sl = pl.ds(k * _CHUNK, _CHUNK)
            descs.append(pltpu.async_copy(pcuml_hbm.at[posid.at[sl]],
                                          valbuf.at[sl], sem_p))
        for d in descs:
            d.wait()

        def update(b, carry):
            sl = pl.ds(b * _L, _L)
            lo = lbuf[sl]
            hi = rbnd[sl]
            mid = (lo + hi) >> 1
            ge = valbuf[sl] >= rbuf[sl]
            rbnd[sl] = jnp.where(ge, mid, hi)
            lbuf[sl] = jnp.where(ge, lo, mid + 1)
            return carry

        lax.fori_loop(0, n_blk, update, 0)

    def finalize(b, carry):
        sl = pl.ds(b * _L, _L)
        posid[sl] = lbuf[sl]
        return carry

    lax.fori_loop(0, n_blk, finalize, 0)

    descs = []
    for k in range(_POS_CHUNKS):
        sl = pl.ds(k * _CHUNK, _CHUNK)
        descs.append(pltpu.async_copy(e0_hbm.at[posid.at[sl]],
                                      src_all.at[sl], sem_p))
        descs.append(pltpu.async_copy(e1_hbm.at[posid.at[sl]],
                                      dst_all.at[sl], sem_p))
    for d in descs:
        d.wait()
    dna.wait()
    dnb.wait()

    def fix(k, carry):
        sl = pl.ds(_POS_PER_W + k * _L, _L)
        vs = src_all[sl]
        vd = dst_all[sl]
        vd1 = vd + 1
        vd1 = jnp.where(vd1 == n_nodes, 0, vd1)
        dst_all[sl] = jnp.where(vd == vs, vd1, vd)
        return carry

    lax.fori_loop(0, _NEG_PER_W // _L, fix, 0)

    def issue(c, b):
        sl = pl.ds(c * _CHUNK, _CHUNK)
        rsl = pl.ds(b * _CHUNK, _CHUNK)
        pltpu.async_copy(node_hbm.at[src_all.at[sl]], rows_i.at[rsl],
                         sems_i.at[b])
        pltpu.async_copy(node_hbm.at[dst_all.at[sl]], rows_j.at[rsl],
                         sems_j.at[b])

    def wait(c, b):
        sl = pl.ds(c * _CHUNK, _CHUNK)
        rsl = pl.ds(b * _CHUNK, _CHUNK)
        pltpu.make_async_copy(node_hbm.at[src_all.at[sl]], rows_i.at[rsl],
                              sems_i.at[b]).wait()
        pltpu.make_async_copy(node_hbm.at[dst_all.at[sl]], rows_j.at[rsl],
                              sems_j.at[b]).wait()

    def chunk_loss(b, is_pos_vec, accs):
        acc_pos, acc_neg = accs

        def blk(kb, accs2):
            acc_pos2, acc_neg2 = accs2
            row = b * _CHUNK + kb * _L + iota
            s = jnp.zeros((_L,), jnp.float32)
            for d in range(_DIM):
                col = (iota + d) & (_DIM - 1)
                df = plsc.load_gather(rows_i, [row, col]) - plsc.load_gather(
                    rows_j, [row, col])
                s = s + df * df
            t = s + 1e-12
            u = _A * jnp.exp(_B * _vlog(t))
            v = jnp.where(is_pos_vec, 1.0 + u, 1.0 + 1.0 / u)
            term = jnp.minimum(_vlog(v), _MAX_TERM)
            zero = jnp.zeros((_L,), jnp.float32)
            acc_pos2 = acc_pos2 + jnp.where(is_pos_vec, term, zero)
            acc_neg2 = acc_neg2 + jnp.where(is_pos_vec, zero, term)
            return acc_pos2, acc_neg2

        return lax.fori_loop(0, _BLKS, blk, (acc_pos, acc_neg))

    for b in range(_NB):
        issue(b, b)

    zero = jnp.zeros((_L,), jnp.float32)

    def outer(g, accs):
        for b in range(_NB):
            c = g * _NB + b
            wait(c, b)
            is_pos_vec = jnp.full((_L,), c, jnp.int32) < _POS_CHUNKS
            accs = chunk_loss(b, is_pos_vec, accs)

            @pl.when(c + _NB < _NCHUNKS)
            def _():
                issue(c + _NB, b)

        return accs

    acc_pos, acc_neg = lax.fori_loop(0, _NCHUNKS // _NB, outer, (zero, zero))
    stage[0] = acc_pos
    stage[1] = acc_neg
    pltpu.sync_copy(stage, out_hbm.at[wid])


@jax.jit
def _sc_loss(node_pos, e0, e1, p_cuml, coarse, r, neg_src, neg_dst):
    mesh = plsc.VectorSubcoreMesh(core_axis_name="c", subcore_axis_name="s")
    n_edges = p_cuml.shape[0]
    n_coarse = (n_edges + _COARSE_STEP - 1) // _COARSE_STEP
    f = functools.partial(
        pl.kernel,
        out_type=jax.ShapeDtypeStruct((_NW, 2, _L), jnp.float32),
        mesh=mesh,
        compiler_params=pltpu.CompilerParams(
            needs_layout_passes=False, use_tc_tiling_on_sc=False),
        scratch_types=[
            pltpu.VMEM((_POS_PER_W,), jnp.int32),
            pltpu.VMEM((_PAIRS_PER_W,), jnp.int32),
            pltpu.VMEM((_PAIRS_PER_W,), jnp.int32),
            pltpu.VMEM((_NB * _CHUNK, _DIM), jnp.float32),
            pltpu.VMEM((_NB * _CHUNK, _DIM), jnp.float32),
            pltpu.VMEM((2, _L), jnp.float32),
            pltpu.VMEM((n_coarse,), jnp.float32),
            pltpu.VMEM((_POS_PER_W,), jnp.float32),
            pltpu.VMEM((_POS_PER_W,), jnp.int32),
            pltpu.VMEM((_POS_PER_W,), jnp.int32),
            pltpu.VMEM((_POS_PER_W,), jnp.float32),
            pltpu.SemaphoreType.DMA,
            pltpu.SemaphoreType.DMA,
            pltpu.SemaphoreType.DMA((_NB,)),
            pltpu.SemaphoreType.DMA((_NB,)),
        ],
    )(_sc_body)
    return f(node_pos, e0, e1, p_cuml, coarse, r, neg_src, neg_dst)


def kernel(node_pos, edge_index, edge_weight):
    n_nodes = node_pos.shape[0]
    n_edges = edge_index.shape[1]
    w = jnp.clip(edge_weight, 1e-12, None)
    p = w / w.sum()
    key = jax.random.key(42)
    kpos, kneg = jax.random.split(key)
    p_cuml = jnp.cumsum(p)
    r = p_cuml[-1] * (1.0 - jax.random.uniform(kpos, (_P,),
                                               dtype=p_cuml.dtype))
    coarse = p_cuml[::_COARSE_STEP]
    kn1, kn2 = jax.random.split(kneg)
    neg_src = jax.random.randint(kn1, (_NNEG,), 0, n_nodes, dtype=jnp.int32)
    neg_dst = jax.random.randint(kn2, (_NNEG,), 0, n_nodes, dtype=jnp.int32)

    parts = _sc_loss(node_pos, edge_index[0], edge_index[1], p_cuml, coarse,
                     r, neg_src, neg_dst)
    attraction = jnp.sum(parts[:, 0, :]) / _P
    repulsion = jnp.sum(parts[:, 1, :]) / _NNEG
    return attraction + _GAMMA * repulsion


# speedup vs baseline: 8.5013x; 1.0780x over previous
import functools

import jax
import jax.numpy as jnp
from jax import lax
from jax.experimental import pallas as pl
from jax.experimental.pallas import tpu as pltpu
from jax.experimental.pallas import tpu_sc as plsc

_A = 1.576943460405378
_B = 0.8950608781227859
_P = 65536
_NEG_PER_EDGE = 5
_GAMMA = 1.0
_DIM = 64

_NC, _NS, _L = 2, 16, 16
_NW = _NC * _NS
_NNEG = _P * _NEG_PER_EDGE
_POS_PER_W = _P // _NW
_NEG_PER_W = _NNEG // _NW
_CHUNK = 128
_POS_CHUNKS = _POS_PER_W // _CHUNK
_NEG_CHUNKS = _NEG_PER_W // _CHUNK
_NB = 4
_BLKS = _CHUNK // _L

_COARSE_STEP = 64
_LOCAL_LEVELS = 14
_HBM_LEVELS = 6

_LN2 = 0.6931471805599453
_MAX_TERM = 9.210340371976184
_SQRT2 = 1.4142135623730951


def _vlog(x):
    bits = plsc.bitcast(x, jnp.int32)
    e = (bits >> 23) - 127
    m = plsc.bitcast((bits & 0x007FFFFF) | 0x3F800000, jnp.float32)
    big = m > _SQRT2
    m = jnp.where(big, m * 0.5, m)
    e = jnp.where(big, e + 1, e)
    z = (m - 1.0) / (m + 1.0)
    z2 = z * z
    poly = 1.0 + z2 * (1.0 / 3.0 + z2 * (1.0 / 5.0 + z2 * (1.0 / 7.0)))
    return e.astype(jnp.float32) * _LN2 + 2.0 * z * poly


def _pair_loss_pipeline(node_hbm, src_all, dst_all, rows_i, rows_j,
                        sems_i, sems_j, n_chunks, is_pos):
    iota = lax.iota(jnp.int32, _L)

    def issue(c, b):
        sl = pl.ds(c * _CHUNK, _CHUNK)
        rsl = pl.ds(b * _CHUNK, _CHUNK)
        pltpu.async_copy(node_hbm.at[src_all.at[sl]], rows_i.at[rsl],
                         sems_i.at[b])
        pltpu.async_copy(node_hbm.at[dst_all.at[sl]], rows_j.at[rsl],
                         sems_j.at[b])

    def wait(c, b):
        sl = pl.ds(c * _CHUNK, _CHUNK)
        rsl = pl.ds(b * _CHUNK, _CHUNK)
        pltpu.make_async_copy(node_hbm.at[src_all.at[sl]], rows_i.at[rsl],
                              sems_i.at[b]).wait()
        pltpu.make_async_copy(node_hbm.at[dst_all.at[sl]], rows_j.at[rsl],
                              sems_j.at[b]).wait()

    def chunk_loss(b, acc):
        def blk(kb, acc2):
            row = b * _CHUNK + kb * _L + iota
            s = jnp.zeros((_L,), jnp.float32)
            for d in range(_DIM):
                col = (iota + d) & (_DIM - 1)
                df = plsc.load_gather(rows_i, [row, col]) - plsc.load_gather(
                    rows_j, [row, col])
                s = s + df * df
            t = s + 1e-12
            u = _A * jnp.exp(_B * _vlog(t))
            v = 1.0 + u if is_pos else 1.0 + 1.0 / u
            return acc2 + jnp.minimum(_vlog(v), _MAX_TERM)

        return lax.fori_loop(0, _BLKS, blk, acc)

    for b in range(_NB):
        issue(b, b)

    def outer(g, acc):
        for b in range(_NB):
            c = g * _NB + b
            wait(c, b)
            acc = chunk_loss(b, acc)

            @pl.when(c + _NB < n_chunks)
            def _():
                issue(c + _NB, b)

        return acc

    zero = jnp.zeros((_L,), jnp.float32)
    return lax.fori_loop(0, n_chunks // _NB, outer, zero)


def _neg_body(node_hbm, neg_src_hbm, neg_dst_hbm, out_hbm,
              src_all, dst_all, rows_i, rows_j, stage, sems_i, sems_j):
    n_nodes = node_hbm.shape[0]
    wid = lax.axis_index("s") * _NC + lax.axis_index("c")

    pltpu.sync_copy(neg_src_hbm.at[pl.ds(wid * _NEG_PER_W, _NEG_PER_W)],
                    src_all)
    pltpu.sync_copy(neg_dst_hbm.at[pl.ds(wid * _NEG_PER_W, _NEG_PER_W)],
                    dst_all)

    def fix(k, carry):
        sl = pl.ds(k * _L, _L)
        vs = src_all[sl]
        vd = dst_all[sl]
        vd1 = vd + 1
        vd1 = jnp.where(vd1 == n_nodes, 0, vd1)
        dst_all[sl] = jnp.where(vd == vs, vd1, vd)
        return carry

    lax.fori_loop(0, _NEG_PER_W // _L, fix, 0)

    acc = _pair_loss_pipeline(node_hbm, src_all, dst_all, rows_i, rows_j,
                              sems_i, sems_j, _NEG_CHUNKS, False)
    stage[0] = acc
    pltpu.sync_copy(stage, out_hbm.at[wid])


def _pos_body(node_hbm, e0_hbm, e1_hbm, pcuml_hbm, coarse_hbm, r_hbm,
              out_hbm, posid, src_all, dst_all, rows_i, rows_j, stage,
              coarse, rbuf, lbuf, rbnd, valbuf, sem_p, sems_i, sems_j):
    n_edges = pcuml_hbm.shape[0]
    n_coarse = (n_edges + _COARSE_STEP - 1) // _COARSE_STEP
    wid = lax.axis_index("s") * _NC + lax.axis_index("c")
    n_blk = _POS_PER_W // _L

    pltpu.sync_copy(coarse_hbm.at[pl.ds(0, n_coarse)], coarse)
    pltpu.sync_copy(r_hbm.at[pl.ds(wid * _POS_PER_W, _POS_PER_W)], rbuf)

    def local_search(b, carry):
        sl = pl.ds(b * _L, _L)
        rv = rbuf[sl]
        lo = jnp.zeros((_L,), jnp.int32)
        hi = jnp.full((_L,), n_coarse - 1, jnp.int32)
        for _ in range(_LOCAL_LEVELS):
            mid = (lo + hi) >> 1
            ge = plsc.load_gather(coarse, [mid]) >= rv
            hi = jnp.where(ge, mid, hi)
            lo = jnp.where(ge, lo, mid + 1)
        jc = jnp.minimum(lo, n_coarse - 1)
        j = jnp.where(plsc.load_gather(coarse, [jc]) >= rv, jc, jc + 1)
        lo0 = jnp.maximum(j * _COARSE_STEP - (_COARSE_STEP - 1), 0)
        hi0 = jnp.minimum(j * _COARSE_STEP, n_edges - 1)
        lbuf[sl] = lo0
        rbnd[sl] = hi0
        return carry

    lax.fori_loop(0, n_blk, local_search, 0)

    for _ in range(_HBM_LEVELS):
        def mids(b, carry):
            sl = pl.ds(b * _L, _L)
            posid[sl] = (lbuf[sl] + rbnd[sl]) >> 1
            return carry

        lax.fori_loop(0, n_blk, mids, 0)
        descs = []
        for k in range(_POS_CHUNKS):
            sl = pl.ds(k * _CHUNK, _CHUNK)
            descs.append(pltpu.async_copy(pcuml_hbm.at[posid.at[sl]],
                                          valbuf.at[sl], sem_p))
        for d in descs:
            d.wait()

        def update(b, carry):
            sl = pl.ds(b * _L, _L)
            lo = lbuf[sl]
            hi = rbnd[sl]
            mid = (lo + hi) >> 1
            ge = valbuf[sl] >= rbuf[sl]
            rbnd[sl] = jnp.where(ge, mid, hi)
            lbuf[sl] = jnp.where(ge, lo, mid + 1)
            return carry

        lax.fori_loop(0, n_blk, update, 0)

    def finalize(b, carry):
        sl = pl.ds(b * _L, _L)
        posid[sl] = lbuf[sl]
        return carry

    lax.fori_loop(0, n_blk, finalize, 0)

    descs = []
    for k in range(_POS_CHUNKS):
        sl = pl.ds(k * _CHUNK, _CHUNK)
        descs.append(pltpu.async_copy(e0_hbm.at[posid.at[sl]],
                                      src_all.at[sl], sem_p))
        descs.append(pltpu.async_copy(e1_hbm.at[posid.at[sl]],
                                      dst_all.at[sl], sem_p))
    for d in descs:
        d.wait()

    acc = _pair_loss_pipeline(node_hbm, src_all, dst_all, rows_i, rows_j,
                              sems_i, sems_j, _POS_CHUNKS, True)
    stage[0] = acc
    pltpu.sync_copy(stage, out_hbm.at[wid])


_SC_PARAMS = pltpu.CompilerParams(
    needs_layout_passes=False, use_tc_tiling_on_sc=False)


@jax.jit
def _sc_neg_loss(node_pos, neg_src, neg_dst):
    mesh = plsc.VectorSubcoreMesh(core_axis_name="c", subcore_axis_name="s")
    f = functools.partial(
        pl.kernel,
        out_type=jax.ShapeDtypeStruct((_NW, 1, _L), jnp.float32),
        mesh=mesh,
        compiler_params=_SC_PARAMS,
        scratch_types=[
            pltpu.VMEM((_NEG_PER_W,), jnp.int32),
            pltpu.VMEM((_NEG_PER_W,), jnp.int32),
            pltpu.VMEM((_NB * _CHUNK, _DIM), jnp.float32),
            pltpu.VMEM((_NB * _CHUNK, _DIM), jnp.float32),
            pltpu.VMEM((1, _L), jnp.float32),
            pltpu.SemaphoreType.DMA((_NB,)),
            pltpu.SemaphoreType.DMA((_NB,)),
        ],
    )(_neg_body)
    return f(node_pos, neg_src, neg_dst)


@jax.jit
def _sc_pos_loss(node_pos, e0, e1, p_cuml, coarse, r):
    mesh = plsc.VectorSubcoreMesh(core_axis_name="c", subcore_axis_name="s")
    n_edges = p_cuml.shape[0]
    n_coarse = (n_edges + _COARSE_STEP - 1) // _COARSE_STEP
    f = functools.partial(
        pl.kernel,
        out_type=jax.ShapeDtypeStruct((_NW, 1, _L), jnp.float32),
        mesh=mesh,
        compiler_params=_SC_PARAMS,
        scratch_types=[
            pltpu.VMEM((_POS_PER_W,), jnp.int32),
            pltpu.VMEM((_POS_PER_W,), jnp.int32),
            pltpu.VMEM((_POS_PER_W,), jnp.int32),
            pltpu.VMEM((_NB * _CHUNK, _DIM), jnp.float32),
            pltpu.VMEM((_NB * _CHUNK, _DIM), jnp.float32),
            pltpu.VMEM((1, _L), jnp.float32),
            pltpu.VMEM((n_coarse,), jnp.float32),
            pltpu.VMEM((_POS_PER_W,), jnp.float32),
            pltpu.VMEM((_POS_PER_W,), jnp.int32),
            pltpu.VMEM((_POS_PER_W,), jnp.int32),
            pltpu.VMEM((_POS_PER_W,), jnp.float32),
            pltpu.SemaphoreType.DMA,
            pltpu.SemaphoreType.DMA((_NB,)),
            pltpu.SemaphoreType.DMA((_NB,)),
        ],
    )(_pos_body)
    return f(node_pos, e0, e1, p_cuml, coarse, r)


def kernel(node_pos, edge_index, edge_weight):
    n_nodes = node_pos.shape[0]
    n_edges = edge_index.shape[1]
    key = jax.random.key(42)
    kpos, kneg = jax.random.split(key)
    kn1, kn2 = jax.random.split(kneg)
    neg_src = jax.random.randint(kn1, (_NNEG,), 0, n_nodes, dtype=jnp.int32)
    neg_dst = jax.random.randint(kn2, (_NNEG,), 0, n_nodes, dtype=jnp.int32)
    neg_parts = _sc_neg_loss(node_pos, neg_src, neg_dst)

    w = jnp.clip(edge_weight, 1e-12, None)
    p = w / w.sum()
    p_cuml = jnp.cumsum(p)
    r = p_cuml[-1] * (1.0 - jax.random.uniform(kpos, (_P,),
                                               dtype=p_cuml.dtype))
    coarse = p_cuml[::_COARSE_STEP]
    pos_parts = _sc_pos_loss(node_pos, edge_index[0], edge_index[1], p_cuml,
                             coarse, r)

    attraction = jnp.sum(pos_parts) / _P
    repulsion = jnp.sum(neg_parts) / _NNEG
    return attraction + _GAMMA * repulsion


# speedup vs baseline: 8.7681x; 1.0314x over previous
import functools

import jax
import jax.numpy as jnp
from jax import lax
from jax.experimental import pallas as pl
from jax.experimental.pallas import tpu as pltpu
from jax.experimental.pallas import tpu_sc as plsc

_A = 1.576943460405378
_B = 0.8950608781227859
_P = 65536
_NEG_PER_EDGE = 5
_GAMMA = 1.0
_DIM = 64

_NC, _NS, _L = 2, 16, 16
_NW = _NC * _NS
_NNEG = _P * _NEG_PER_EDGE
_POS_PER_W = _P // _NW
_NEG_PER_W = _NNEG // _NW
_CHUNK = 128
_POS_CHUNKS = _POS_PER_W // _CHUNK
_NEG_CHUNKS = _NEG_PER_W // _CHUNK
_NB = 4
_BLKS = _CHUNK // _L

_COARSE_STEP = 64
_LOCAL_LEVELS = 14
_HBM_LEVELS = 6

_LN2 = 0.6931471805599453
_MAX_TERM = 9.210340371976184
_SQRT2 = 1.4142135623730951


def _vlog(x):
    bits = plsc.bitcast(x, jnp.int32)
    e = (bits >> 23) - 127
    m = plsc.bitcast((bits & 0x007FFFFF) | 0x3F800000, jnp.float32)
    big = m > _SQRT2
    m = jnp.where(big, m * 0.5, m)
    e = jnp.where(big, e + 1, e)
    z = (m - 1.0) / (m + 1.0)
    z2 = z * z
    poly = 1.0 + z2 * (1.0 / 3.0 + z2 * (1.0 / 5.0 + z2 * (1.0 / 7.0)))
    return e.astype(jnp.float32) * _LN2 + 2.0 * z * poly


def _pair_loss_pipeline(node_hbm, src_all, dst_all, rows_i, rows_j,
                        sems_i, sems_j, n_chunks, is_pos):
    iota = lax.iota(jnp.int32, _L)

    def issue(c, b):
        sl = pl.ds(c * _CHUNK, _CHUNK)
        rsl = pl.ds(b * _CHUNK, _CHUNK)
        pltpu.async_copy(node_hbm.at[src_all.at[sl]], rows_i.at[rsl],
                         sems_i.at[b])
        pltpu.async_copy(node_hbm.at[dst_all.at[sl]], rows_j.at[rsl],
                         sems_j.at[b])

    def wait(c, b):
        sl = pl.ds(c * _CHUNK, _CHUNK)
        rsl = pl.ds(b * _CHUNK, _CHUNK)
        pltpu.make_async_copy(node_hbm.at[src_all.at[sl]], rows_i.at[rsl],
                              sems_i.at[b]).wait()
        pltpu.make_async_copy(node_hbm.at[dst_all.at[sl]], rows_j.at[rsl],
                              sems_j.at[b]).wait()

    def chunk_loss(b, acc):
        def blk(kb, acc2):
            row = b * _CHUNK + kb * _L + iota
            s = jnp.zeros((_L,), jnp.float32)
            for d in range(_DIM):
                col = (iota + d) & (_DIM - 1)
                df = plsc.load_gather(rows_i, [row, col]) - plsc.load_gather(
                    rows_j, [row, col])
                s = s + df * df
            t = s + 1e-12
            u = _A * jnp.exp(_B * _vlog(t))
            v = 1.0 + u if is_pos else 1.0 + 1.0 / u
            return acc2 + jnp.minimum(_vlog(v), _MAX_TERM)

        return lax.fori_loop(0, _BLKS, blk, acc)

    for b in range(_NB):
        issue(b, b)

    def outer(g, acc):
        for b in range(_NB):
            c = g * _NB + b
            wait(c, b)
            acc = chunk_loss(b, acc)

            @pl.when(c + _NB < n_chunks)
            def _():
                issue(c + _NB, b)

        return acc

    zero = jnp.zeros((_L,), jnp.float32)
    return lax.fori_loop(0, n_chunks // _NB, outer, zero)


def _neg_body(node_hbm, neg_src_hbm, neg_dst_hbm, out_hbm,
              src_all, dst_all, rows_i, rows_j, stage, sems_i, sems_j):
    n_nodes = node_hbm.shape[0]
    wid = lax.axis_index("s") * _NC + lax.axis_index("c")

    pltpu.sync_copy(neg_src_hbm.at[pl.ds(wid * _NEG_PER_W, _NEG_PER_W)],
                    src_all)
    pltpu.sync_copy(neg_dst_hbm.at[pl.ds(wid * _NEG_PER_W, _NEG_PER_W)],
                    dst_all)

    def fix(k, carry):
        sl = pl.ds(k * _L, _L)
        vs = src_all[sl]
        vd = dst_all[sl]
        vd1 = vd + 1
        vd1 = jnp.where(vd1 == n_nodes, 0, vd1)
        dst_all[sl] = jnp.where(vd == vs, vd1, vd)
        return carry

    lax.fori_loop(0, _NEG_PER_W // _L, fix, 0)

    acc = _pair_loss_pipeline(node_hbm, src_all, dst_all, rows_i, rows_j,
                              sems_i, sems_j, _NEG_CHUNKS, False)
    stage[0] = acc
    pltpu.sync_copy(stage, out_hbm.at[wid])


def _pos_body(node_hbm, e0_hbm, e1_hbm, pcuml_hbm, coarse_hbm, r_hbm,
              out_hbm, posid, src_all, dst_all, rows_i, rows_j, stage,
              coarse, rbuf, lbuf, rbnd, valbuf, sem_p, sems_i, sems_j):
    n_edges = pcuml_hbm.shape[0]
    n_coarse = (n_edges + _COARSE_STEP - 1) // _COARSE_STEP
    wid = lax.axis_index("s") * _NC + lax.axis_index("c")
    n_blk = _POS_PER_W // _L

    pltpu.sync_copy(coarse_hbm.at[pl.ds(0, n_coarse)], coarse)
    pltpu.sync_copy(r_hbm.at[pl.ds(wid * _POS_PER_W, _POS_PER_W)], rbuf)

    def local_search(b, carry):
        sl = pl.ds(b * _L, _L)
        rv = rbuf[sl]
        lo = jnp.zeros((_L,), jnp.int32)
        hi = jnp.full((_L,), n_coarse - 1, jnp.int32)
        for _ in range(_LOCAL_LEVELS):
            mid = (lo + hi) >> 1
            ge = plsc.load_gather(coarse, [mid]) >= rv
            hi = jnp.where(ge, mid, hi)
            lo = jnp.where(ge, lo, mid + 1)
        jc = jnp.minimum(lo, n_coarse - 1)
        j = jnp.where(plsc.load_gather(coarse, [jc]) >= rv, jc, jc + 1)
        lo0 = jnp.maximum(j * _COARSE_STEP - (_COARSE_STEP - 1), 0)
        hi0 = jnp.minimum(j * _COARSE_STEP, n_edges - 1)
        lbuf[sl] = lo0
        rbnd[sl] = hi0
        return carry

    lax.fori_loop(0, n_blk, local_search, 0)

    for _ in range(_HBM_LEVELS):
        def mids(b, carry):
            sl = pl.ds(b * _L, _L)
            posid[sl] = (lbuf[sl] + rbnd[sl]) >> 1
            return carry

        lax.fori_loop(0, n_blk, mids, 0)
        descs = []
        for k in range(_POS_CHUNKS):
            sl = pl.ds(k * _CHUNK, _CHUNK)
            descs.append(pltpu.async_copy(pcuml_hbm.at[posid.at[sl]],
                                          valbuf.at[sl], sem_p))
        for d in descs:
            d.wait()

        def update(b, carry):
            sl = pl.ds(b * _L, _L)
            lo = lbuf[sl]
            hi = rbnd[sl]
            mid = (lo + hi) >> 1
            ge = valbuf[sl] >= rbuf[sl]
            rbnd[sl] = jnp.where(ge, mid, hi)
            lbuf[sl] = jnp.where(ge, lo, mid + 1)
            return carry

        lax.fori_loop(0, n_blk, update, 0)

    def finalize(b, carry):
        sl = pl.ds(b * _L, _L)
        posid[sl] = lbuf[sl]
        return carry

    lax.fori_loop(0, n_blk, finalize, 0)

    descs = []
    for k in range(_POS_CHUNKS):
        sl = pl.ds(k * _CHUNK, _CHUNK)
        descs.append(pltpu.async_copy(e0_hbm.at[posid.at[sl]],
                                      src_all.at[sl], sem_p))
        descs.append(pltpu.async_copy(e1_hbm.at[posid.at[sl]],
                                      dst_all.at[sl], sem_p))
    for d in descs:
        d.wait()

    acc = _pair_loss_pipeline(node_hbm, src_all, dst_all, rows_i, rows_j,
                              sems_i, sems_j, _POS_CHUNKS, True)
    stage[0] = acc
    pltpu.sync_copy(stage, out_hbm.at[wid])


_SC_PARAMS = pltpu.CompilerParams(
    needs_layout_passes=False, use_tc_tiling_on_sc=False)


@jax.jit
def _sc_neg_loss(node_pos, neg_src, neg_dst):
    mesh = plsc.VectorSubcoreMesh(core_axis_name="c", subcore_axis_name="s")
    f = functools.partial(
        pl.kernel,
        out_type=jax.ShapeDtypeStruct((_NW, 1, _L), jnp.float32),
        mesh=mesh,
        compiler_params=_SC_PARAMS,
        scratch_types=[
            pltpu.VMEM((_NEG_PER_W,), jnp.int32),
            pltpu.VMEM((_NEG_PER_W,), jnp.int32),
            pltpu.VMEM((_NB * _CHUNK, _DIM), jnp.float32),
            pltpu.VMEM((_NB * _CHUNK, _DIM), jnp.float32),
            pltpu.VMEM((1, _L), jnp.float32),
            pltpu.SemaphoreType.DMA((_NB,)),
            pltpu.SemaphoreType.DMA((_NB,)),
        ],
    )(_neg_body)
    return f(node_pos, neg_src, neg_dst)


@jax.jit
def _sc_pos_loss(node_pos, e0, e1, p_cuml, coarse, r):
    mesh = plsc.VectorSubcoreMesh(core_axis_name="c", subcore_axis_name="s")
    n_edges = p_cuml.shape[0]
    n_coarse = (n_edges + _COARSE_STEP - 1) // _COARSE_STEP
    f = functools.partial(
        pl.kernel,
        out_type=jax.ShapeDtypeStruct((_NW, 1, _L), jnp.float32),
        mesh=mesh,
        compiler_params=_SC_PARAMS,
        scratch_types=[
            pltpu.VMEM((_POS_PER_W,), jnp.int32),
            pltpu.VMEM((_POS_PER_W,), jnp.int32),
            pltpu.VMEM((_POS_PER_W,), jnp.int32),
            pltpu.VMEM((_NB * _CHUNK, _DIM), jnp.float32),
            pltpu.VMEM((_NB * _CHUNK, _DIM), jnp.float32),
            pltpu.VMEM((1, _L), jnp.float32),
            pltpu.VMEM((n_coarse,), jnp.float32),
            pltpu.VMEM((_POS_PER_W,), jnp.float32),
            pltpu.VMEM((_POS_PER_W,), jnp.int32),
            pltpu.VMEM((_POS_PER_W,), jnp.int32),
            pltpu.VMEM((_POS_PER_W,), jnp.float32),
            pltpu.SemaphoreType.DMA,
            pltpu.SemaphoreType.DMA((_NB,)),
            pltpu.SemaphoreType.DMA((_NB,)),
        ],
    )(_pos_body)
    return f(node_pos, e0, e1, p_cuml, coarse, r)


def kernel(node_pos, edge_index, edge_weight):
    n_nodes = node_pos.shape[0]
    n_edges = edge_index.shape[1]
    key = jax.random.key(42)
    kpos, kneg = jax.random.split(key)
    kn1, kn2 = jax.random.split(kneg)
    neg_src = jax.random.randint(kn1, (_NNEG,), 0, n_nodes, dtype=jnp.int32)
    neg_dst = jax.random.randint(kn2, (_NNEG,), 0, n_nodes, dtype=jnp.int32)
    neg_parts = _sc_neg_loss(node_pos, neg_src, neg_dst)

    w = jnp.clip(edge_weight, 1e-12, None)
    p_cuml = jnp.cumsum(w)
    r = p_cuml[-1] * (1.0 - jax.random.uniform(kpos, (_P,),
                                               dtype=p_cuml.dtype))
    coarse = p_cuml[::_COARSE_STEP]
    pos_parts = _sc_pos_loss(node_pos, edge_index[0], edge_index[1], p_cuml,
                             coarse, r)

    attraction = jnp.sum(pos_parts) / _P
    repulsion = jnp.sum(neg_parts) / _NNEG
    return attraction + _GAMMA * repulsion
